# Initial kernel scaffold; baseline (speedup 1.0000x reference)
import jax
import jax.numpy as jnp
from jax.experimental import pallas as pl


def kernel(features, points, neighbors, W1, b1, kernel_points, Wkp, b_conv, W2, b2, Wsc, bsc):
    raise NotImplementedError("write your pallas kernel here")


# speedup vs baseline: 1.5900x; 1.5900x over previous
import functools

import jax
import jax.numpy as jnp
from jax import lax
from jax.experimental import pallas as pl
from jax.experimental.pallas import tpu as pltpu
from jax.experimental.pallas import tpu_sc as plsc

N = 10000
NEIGH = 32
IN_DIM = 128
OUT_DIM = 256
MID = 64
K = 15
KP_EXTENT = 1.2
TW = 80
BN = 400
NBLK = N // BN

_SC = plsc.get_sparse_core_info()
_NC = _SC.num_cores
_NS = _SC.num_subcores
_NW = _NC * _NS
_ROWS = N * NEIGH
_RPW = _ROWS // _NW
_CHUNK = 1000


def _leaky(x):
    return jnp.where(x >= 0, x, 0.1 * x)


def _table_kernel(feat_ref, pts_ref, w1_ref, b1_ref, out_ref):
    x = jnp.dot(feat_ref[...], w1_ref[...], preferred_element_type=jnp.float32)
    x = _leaky(x + b1_ref[...])
    pts = pts_ref[...]
    pn2 = jnp.sum(pts * pts, axis=1, keepdims=True)
    pad = jnp.zeros((N, TW - MID - 4), dtype=jnp.float32)
    out_ref[...] = jnp.concatenate([x, pts, pn2, pad], axis=1)


def _build_table(features, points, W1, b1):
    return pl.pallas_call(
        _table_kernel,
        out_shape=jax.ShapeDtypeStruct((N, TW), jnp.float32),
    )(features, points, W1, b1.reshape(1, MID))


def _sc_gather(table, idx_flat):
    mesh = plsc.VectorSubcoreMesh(core_axis_name="c", subcore_axis_name="s")

    @functools.partial(
        pl.kernel,
        mesh=mesh,
        compiler_params=pltpu.CompilerParams(use_tc_tiling_on_sc=False),
        out_type=jax.ShapeDtypeStruct((_ROWS, TW), jnp.float32),
        scratch_types=[
            pltpu.VMEM((_CHUNK,), jnp.int32),
            pltpu.VMEM((_CHUNK, TW), jnp.float32),
            pltpu.SemaphoreType.DMA,
        ],
    )
    def gather_k(table_hbm, idx_hbm, out_hbm, idx_v, rows_v, sem):
        wid = lax.axis_index("s") * _NC + lax.axis_index("c")
        base = wid * _RPW

        def body(i, _):
            off = base + i * _CHUNK
            pltpu.sync_copy(idx_hbm.at[pl.ds(off, _CHUNK)], idx_v)
            pltpu.async_copy(table_hbm.at[idx_v], rows_v, sem).wait()
            pltpu.sync_copy(rows_v, out_hbm.at[pl.ds(off, _CHUNK)])
            return 0

        lax.fori_loop(0, _RPW // _CHUNK, body, 0)

    return gather_k(table, idx_flat)


def _block_kernel(g_ref, feat_ref, pts_ref, kp2t_ref, kpn2_ref, wkp_ref,
                  bconv_ref, w2_ref, b2_ref, wsc_ref, bsc_ref, out_ref):
    M = BN * NEIGH
    g = g_ref[...]
    nx = g[:, :MID]
    pn = g[:, MID:MID + 3]
    q = pts_ref[...]

    c = pn.reshape(BN, NEIGH, 3) - q[:, None, :]
    cf = c.reshape(M, 3)
    cn2 = jnp.sum(cf * cf, axis=1, keepdims=True)
    d2 = cn2 - jnp.dot(cf, kp2t_ref[...],
                       preferred_element_type=jnp.float32) + kpn2_ref[...]
    d2 = jnp.maximum(d2, 0.0)
    w = jnp.maximum(1.0 - jnp.sqrt(d2) * (1.0 / KP_EXTENT), 0.0)

    aggs = []
    for k in range(K):
        yk = w[:, k:k + 1] * nx
        aggs.append(jnp.sum(yk.reshape(BN, NEIGH, MID), axis=1))
    agg = jnp.concatenate(aggs, axis=1)

    xkp = jnp.dot(agg, wkp_ref[...], preferred_element_type=jnp.float32)
    x2 = _leaky(xkp + bconv_ref[...])
    x3 = jnp.dot(x2, w2_ref[...], preferred_element_type=jnp.float32) + b2_ref[...]
    sc = jnp.dot(feat_ref[...], wsc_ref[...],
                 preferred_element_type=jnp.float32) + bsc_ref[...]
    out_ref[...] = _leaky(x3 + sc)


def _stage_c(g, features, points, kp2t, kpn2, wkp_flat, b_conv, W2, b2, Wsc, bsc):
    full = lambda shape: pl.BlockSpec(shape, lambda i: (0, 0))
    return pl.pallas_call(
        _block_kernel,
        grid=(NBLK,),
        in_specs=[
            pl.BlockSpec((BN * NEIGH, TW), lambda i: (i, 0)),
            pl.BlockSpec((BN, IN_DIM), lambda i: (i, 0)),
            pl.BlockSpec((BN, 3), lambda i: (i, 0)),
            full((3, K)),
            full((1, K)),
            full((K * MID, MID)),
            full((1, MID)),
            full((MID, OUT_DIM)),
            full((1, OUT_DIM)),
            full((IN_DIM, OUT_DIM)),
            full((1, OUT_DIM)),
        ],
        out_specs=pl.BlockSpec((BN, OUT_DIM), lambda i: (i, 0)),
        out_shape=jax.ShapeDtypeStruct((N, OUT_DIM), jnp.float32),
    )(g, features, points, kp2t, kpn2, wkp_flat, b_conv, W2, b2, Wsc, bsc)


def kernel(features, points, neighbors, W1, b1, kernel_points, Wkp, b_conv,
           W2, b2, Wsc, bsc):
    table = _build_table(features, points, W1, b1)
    g = _sc_gather(table, neighbors.reshape(_ROWS))
    kp2t = 2.0 * kernel_points.T
    kpn2 = jnp.sum(kernel_points * kernel_points, axis=1).reshape(1, K)
    wkp_flat = Wkp.reshape(K * MID, MID)
    return _stage_c(g, features, points, kp2t, kpn2, wkp_flat,
                    b_conv.reshape(1, MID), W2, b2.reshape(1, OUT_DIM),
                    Wsc, bsc.reshape(1, OUT_DIM))


# speedup vs baseline: 3.5959x; 2.2615x over previous
import functools

import jax
import jax.numpy as jnp
from jax import lax
from jax.experimental import pallas as pl
from jax.experimental.pallas import tpu as pltpu
from jax.experimental.pallas import tpu_sc as plsc

N = 10000
NEIGH = 32
IN_DIM = 128
OUT_DIM = 256
MID = 64
K = 15
KP_EXTENT = 1.2
TW = 80
BN = 400
NBLK = N // BN

_SC = plsc.get_sparse_core_info()
_NC = _SC.num_cores
_NS = _SC.num_subcores
_NW = _NC * _NS
_ROWS = N * NEIGH
_RPW = _ROWS // _NW
_CHUNK = 1000


def _leaky(x):
    return jnp.where(x >= 0, x, 0.1 * x)


def _table_kernel(feat_ref, pts_ref, w1_ref, b1_ref, out_ref):
    x = jnp.dot(feat_ref[...], w1_ref[...], preferred_element_type=jnp.float32)
    x = _leaky(x + b1_ref[...])
    pts = pts_ref[...]
    pn2 = jnp.sum(pts * pts, axis=1, keepdims=True)
    pad = jnp.zeros((N, TW - MID - 4), dtype=jnp.float32)
    out_ref[...] = jnp.concatenate([x, pts, pn2, pad], axis=1)


def _build_table(features, points, W1, b1):
    return pl.pallas_call(
        _table_kernel,
        out_shape=jax.ShapeDtypeStruct((N, TW), jnp.float32),
    )(features, points, W1, b1.reshape(1, MID))


def _sc_gather(table, idx_flat):
    mesh = plsc.VectorSubcoreMesh(core_axis_name="c", subcore_axis_name="s")

    @functools.partial(
        pl.kernel,
        mesh=mesh,
        compiler_params=pltpu.CompilerParams(use_tc_tiling_on_sc=False),
        out_type=jax.ShapeDtypeStruct((_ROWS, TW), jnp.float32),
        scratch_types=[
            pltpu.VMEM((_CHUNK,), jnp.int32),
            pltpu.VMEM((_CHUNK, TW), jnp.float32),
            pltpu.SemaphoreType.DMA,
        ],
    )
    def gather_k(table_hbm, idx_hbm, out_hbm, idx_v, rows_v, sem):
        wid = lax.axis_index("s") * _NC + lax.axis_index("c")
        base = wid * _RPW

        def body(i, _):
            off = base + i * _CHUNK
            pltpu.sync_copy(idx_hbm.at[pl.ds(off, _CHUNK)], idx_v)
            pltpu.async_copy(table_hbm.at[idx_v], rows_v, sem).wait()
            pltpu.sync_copy(rows_v, out_hbm.at[pl.ds(off, _CHUNK)])
            return 0

        lax.fori_loop(0, _RPW // _CHUNK, body, 0)

    return gather_k(table, idx_flat)


def _block_kernel(g_ref, feat_ref, pts_ref, kp2t_ref, kpn2_ref, wkp_ref,
                  bconv_ref, w2_ref, b2_ref, wsc_ref, bsc_ref, out_ref):
    M = BN * NEIGH
    g = g_ref[...]
    nx = g[:, :MID]
    pn = g[:, MID:MID + 3]
    q = pts_ref[...]

    c = pn.reshape(BN, NEIGH, 3) - q[:, None, :]
    cf = c.reshape(M, 3)
    cn2 = jnp.sum(cf * cf, axis=1, keepdims=True)
    d2 = cn2 - jnp.dot(cf, kp2t_ref[...],
                       preferred_element_type=jnp.float32) + kpn2_ref[...]
    d2 = jnp.maximum(d2, 0.0)
    w = jnp.maximum(1.0 - jnp.sqrt(d2) * (1.0 / KP_EXTENT), 0.0)

    wr = w.reshape(BN, NEIGH, K)
    nxr = nx.reshape(BN, NEIGH, MID)
    agg = lax.dot_general(wr, nxr, (((1,), (1,)), ((0,), (0,))),
                          preferred_element_type=jnp.float32)
    agg = agg.reshape(BN, K * MID)

    xkp = jnp.dot(agg, wkp_ref[...], preferred_element_type=jnp.float32)
    x2 = _leaky(xkp + bconv_ref[...])
    x3 = jnp.dot(x2, w2_ref[...], preferred_element_type=jnp.float32) + b2_ref[...]
    sc = jnp.dot(feat_ref[...], wsc_ref[...],
                 preferred_element_type=jnp.float32) + bsc_ref[...]
    out_ref[...] = _leaky(x3 + sc)


def _stage_c(g, features, points, kp2t, kpn2, wkp_flat, b_conv, W2, b2, Wsc, bsc):
    full = lambda shape: pl.BlockSpec(shape, lambda i: (0, 0))
    return pl.pallas_call(
        _block_kernel,
        grid=(NBLK,),
        in_specs=[
            pl.BlockSpec((BN * NEIGH, TW), lambda i: (i, 0)),
            pl.BlockSpec((BN, IN_DIM), lambda i: (i, 0)),
            pl.BlockSpec((BN, 3), lambda i: (i, 0)),
            full((3, K)),
            full((1, K)),
            full((K * MID, MID)),
            full((1, MID)),
            full((MID, OUT_DIM)),
            full((1, OUT_DIM)),
            full((IN_DIM, OUT_DIM)),
            full((1, OUT_DIM)),
        ],
        out_specs=pl.BlockSpec((BN, OUT_DIM), lambda i: (i, 0)),
        out_shape=jax.ShapeDtypeStruct((N, OUT_DIM), jnp.float32),
    )(g, features, points, kp2t, kpn2, wkp_flat, b_conv, W2, b2, Wsc, bsc)


def kernel(features, points, neighbors, W1, b1, kernel_points, Wkp, b_conv,
           W2, b2, Wsc, bsc):
    table = _build_table(features, points, W1, b1)
    g = _sc_gather(table, neighbors.reshape(_ROWS))
    kp2t = 2.0 * kernel_points.T
    kpn2 = jnp.sum(kernel_points * kernel_points, axis=1).reshape(1, K)
    wkp_flat = Wkp.reshape(K * MID, MID)
    return _stage_c(g, features, points, kp2t, kpn2, wkp_flat,
                    b_conv.reshape(1, MID), W2, b2.reshape(1, OUT_DIM),
                    Wsc, bsc.reshape(1, OUT_DIM))


# speedup vs baseline: 4.4732x; 1.2440x over previous
import functools

import jax
import jax.numpy as jnp
from jax import lax
from jax.experimental import pallas as pl
from jax.experimental.pallas import tpu as pltpu
from jax.experimental.pallas import tpu_sc as plsc

N = 10000
NEIGH = 32
IN_DIM = 128
OUT_DIM = 256
MID = 64
K = 15
KP_EXTENT = 1.2
TW = 128
BN = 400
NBLK = N // BN

_SC = plsc.get_sparse_core_info()
_NC = _SC.num_cores
_NS = _SC.num_subcores
_NW = _NC * _NS
_ROWS = N * NEIGH
_RPW = _ROWS // _NW
_CHUNK = 400


def _leaky(x):
    return jnp.where(x >= 0, x, 0.1 * x)


def _table_kernel(feat_ref, pts_ref, w1_ref, b1_ref, out_ref):
    x = jnp.dot(feat_ref[...], w1_ref[...], preferred_element_type=jnp.float32)
    x = _leaky(x + b1_ref[...])
    pts = pts_ref[...]
    pn2 = jnp.sum(pts * pts, axis=1, keepdims=True)
    pad = jnp.zeros((N, TW - MID - 4), dtype=jnp.float32)
    out_ref[...] = jnp.concatenate([x, pts, pn2, pad], axis=1)


def _build_table(features, points, W1, b1):
    return pl.pallas_call(
        _table_kernel,
        out_shape=jax.ShapeDtypeStruct((N, TW), jnp.float32),
    )(features, points, W1, b1.reshape(1, MID))


def _sc_gather(table, idx_flat):
    mesh = plsc.VectorSubcoreMesh(core_axis_name="c", subcore_axis_name="s")

    @functools.partial(
        pl.kernel,
        mesh=mesh,
        out_type=jax.ShapeDtypeStruct((_ROWS, TW), jnp.float32),
        scratch_types=[
            pltpu.VMEM((_CHUNK,), jnp.int32),
            pltpu.VMEM((_CHUNK, TW), jnp.float32),
            pltpu.SemaphoreType.DMA,
        ],
    )
    def gather_k(table_hbm, idx_hbm, out_hbm, idx_v, rows_v, sem):
        wid = lax.axis_index("s") * _NC + lax.axis_index("c")
        base = wid * _RPW

        def body(i, _):
            off = base + i * _CHUNK
            pltpu.sync_copy(idx_hbm.at[pl.ds(off, _CHUNK)], idx_v)
            pltpu.async_copy(table_hbm.at[idx_v], rows_v, sem).wait()
            pltpu.sync_copy(rows_v, out_hbm.at[pl.ds(off, _CHUNK)])
            return 0

        lax.fori_loop(0, _RPW // _CHUNK, body, 0)

    return gather_k(table, idx_flat)


def _block_kernel(g_ref, feat_ref, pts_ref, kp2t_ref, kpn2_ref, wkp_ref,
                  bconv_ref, w2_ref, b2_ref, wsc_ref, bsc_ref, out_ref):
    M = BN * NEIGH
    g = g_ref[...]
    nx = g[:, :MID]
    pn = g[:, MID:MID + 3]
    q = pts_ref[...]

    c = pn.reshape(BN, NEIGH, 3) - q[:, None, :]
    cf = c.reshape(M, 3)
    cn2 = jnp.sum(cf * cf, axis=1, keepdims=True)
    d2 = cn2 - jnp.dot(cf, kp2t_ref[...],
                       preferred_element_type=jnp.float32) + kpn2_ref[...]
    d2 = jnp.maximum(d2, 0.0)
    w = jnp.maximum(1.0 - jnp.sqrt(d2) * (1.0 / KP_EXTENT), 0.0)

    wr = w.reshape(BN, NEIGH, K)
    nxr = nx.reshape(BN, NEIGH, MID)
    agg = lax.dot_general(wr, nxr, (((1,), (1,)), ((0,), (0,))),
                          preferred_element_type=jnp.float32)
    agg = agg.reshape(BN, K * MID)

    xkp = jnp.dot(agg, wkp_ref[...], preferred_element_type=jnp.float32)
    x2 = _leaky(xkp + bconv_ref[...])
    x3 = jnp.dot(x2, w2_ref[...], preferred_element_type=jnp.float32) + b2_ref[...]
    sc = jnp.dot(feat_ref[...], wsc_ref[...],
                 preferred_element_type=jnp.float32) + bsc_ref[...]
    out_ref[...] = _leaky(x3 + sc)


def _stage_c(g, features, points, kp2t, kpn2, wkp_flat, b_conv, W2, b2, Wsc, bsc):
    full = lambda shape: pl.BlockSpec(shape, lambda i: (0, 0))
    return pl.pallas_call(
        _block_kernel,
        grid=(NBLK,),
        in_specs=[
            pl.BlockSpec((BN * NEIGH, TW), lambda i: (i, 0)),
            pl.BlockSpec((BN, IN_DIM), lambda i: (i, 0)),
            pl.BlockSpec((BN, 3), lambda i: (i, 0)),
            full((3, K)),
            full((1, K)),
            full((K * MID, MID)),
            full((1, MID)),
            full((MID, OUT_DIM)),
            full((1, OUT_DIM)),
            full((IN_DIM, OUT_DIM)),
            full((1, OUT_DIM)),
        ],
        out_specs=pl.BlockSpec((BN, OUT_DIM), lambda i: (i, 0)),
        out_shape=jax.ShapeDtypeStruct((N, OUT_DIM), jnp.float32),
    )(g, features, points, kp2t, kpn2, wkp_flat, b_conv, W2, b2, Wsc, bsc)


def kernel(features, points, neighbors, W1, b1, kernel_points, Wkp, b_conv,
           W2, b2, Wsc, bsc):
    table = _build_table(features, points, W1, b1)
    g = _sc_gather(table, neighbors.reshape(_ROWS))
    kp2t = 2.0 * kernel_points.T
    kpn2 = jnp.sum(kernel_points * kernel_points, axis=1).reshape(1, K)
    wkp_flat = Wkp.reshape(K * MID, MID)
    return _stage_c(g, features, points, kp2t, kpn2, wkp_flat,
                    b_conv.reshape(1, MID), W2, b2.reshape(1, OUT_DIM),
                    Wsc, bsc.reshape(1, OUT_DIM))
